# baseline (speedup 1.0000x reference)
import functools
import math

import jax
import jax.numpy as jnp
from jax import lax
from jax.experimental import pallas as pl
from jax.experimental.pallas import tpu as pltpu
from jax.experimental.pallas import tpu_sc as plsc

NUM_CORES = 2
NUM_SUBCORES = 16
NUM_WORKERS = NUM_CORES * NUM_SUBCORES
LANES = 16
BLOCK = 2000

_C0 = math.sqrt(1.0 / (4.0 * math.pi))
_C1 = math.sqrt(3.0 / (4.0 * math.pi))
_C2M2 = math.sqrt(15.0 / (4.0 * math.pi))
_C20 = 0.25 * math.sqrt(5.0 / math.pi)
_C22 = 0.25 * math.sqrt(15.0 / math.pi)
_C3M3 = math.sqrt(35.0 / (32.0 * math.pi))
_C3M2 = 0.5 * math.sqrt(105.0 / math.pi)
_C3M1 = math.sqrt(21.0 / (32.0 * math.pi))
_C30 = 0.25 * math.sqrt(7.0 / math.pi)
_C32 = 0.25 * math.sqrt(105.0 / math.pi)


def _splat_f(v):
    return jnp.full((LANES,), v, jnp.float32)


def _splat_i(v):
    return jnp.full((LANES,), v, jnp.int32)


def _rsqrt_newton(s2):
    i = lax.bitcast_convert_type(s2, jnp.int32)
    seed = _splat_i(0x5F3759DF) - lax.shift_right_arithmetic(i, _splat_i(1))
    y = lax.bitcast_convert_type(seed, jnp.float32)
    half = _splat_f(0.5) * s2
    three_half = _splat_f(1.5)
    for _ in range(2):
        y = y * (three_half - half * y * y)
    return y


def _sh_coeffs(X, Y, Z):
    X2 = X * X
    Y2 = Y * Y
    Z2 = Z * Z
    XY = X * Y
    one = _splat_f(1.0)
    return [
        _splat_f(_C0),
        _splat_f(-_C1) * Y,
        _splat_f(_C1) * Z,
        _splat_f(-_C1) * X,
        _splat_f(_C2M2) * XY,
        _splat_f(-_C2M2) * (Y * Z),
        _splat_f(_C20) * (_splat_f(3.0) * Z2 - one),
        _splat_f(-_C2M2) * (X * Z),
        _splat_f(_C22) * (X2 - Y2),
        _splat_f(-_C3M3) * Y * (_splat_f(3.0) * X2 - Y2),
        _splat_f(_C3M2) * XY * Z,
        _splat_f(-_C3M1) * Y * (_splat_f(5.0) * Z2 - one),
        _splat_f(_C30) * Z * (_splat_f(5.0) * Z2 - _splat_f(3.0)),
        _splat_f(-_C3M1) * X * (_splat_f(5.0) * Z2 - one),
        _splat_f(_C32) * Z * (X2 - Y2),
        _splat_f(-_C3M3) * X * (X2 - _splat_f(3.0) * Y2),
    ]


def kernel(xyz_data, xyz_query, nn_idx):
    num_edges = nn_idx.shape[0]
    per_worker = num_edges // NUM_WORKERS
    assert per_worker * NUM_WORKERS == num_edges
    assert per_worker % BLOCK == 0
    nblocks = per_worker // BLOCK
    assert nblocks >= 3 and nblocks % 2 == 1

    xd, yd, zd = (xyz_data[:, c] for c in range(3))
    xq, yq, zq = (xyz_query[:, c] for c in range(3))
    nn_flat = nn_idx.reshape(-1)

    mesh = plsc.VectorSubcoreMesh(core_axis_name="c", subcore_axis_name="s")

    scratch = (
        [pltpu.VMEM((2 * BLOCK,), jnp.int32)] * 2
        + [pltpu.VMEM((BLOCK,), jnp.int32)] * 4
        + [pltpu.VMEM((BLOCK,), jnp.float32)] * 12
        + [pltpu.VMEM((BLOCK, 16), jnp.float32)] * 2
        + [pltpu.SemaphoreType.DMA] * 4
    )

    @functools.partial(
        pl.kernel,
        out_type=jax.ShapeDtypeStruct((num_edges, 16), jnp.float32),
        mesh=mesh,
        scratch_types=scratch,
        compiler_params=pltpu.CompilerParams(
            needs_layout_passes=False, use_tc_tiling_on_sc=False
        ),
    )
    def sc_kernel(
        xd_hbm, yd_hbm, zd_hbm, xq_hbm, yq_hbm, zq_hbm, nn_hbm, out_hbm,
        nn0, nn1,
        iq0, iq1, id0, id1,
        xd0, xd1, yd0, yd1, zd0, zd1, xq0, xq1, yq0, yq1, zq0, zq1,
        ov0, ov1,
        sg0, sg1, so0, so1,
    ):
        wid = lax.axis_index("s") * NUM_CORES + lax.axis_index("c")
        lane = lax.iota(jnp.int32, 16)
        nn_v = (nn0, nn1)
        iq_v = (iq0, iq1)
        id_v = (id0, id1)
        planes = ((xd0, xd1), (yd0, yd1), (zd0, zd1),
                  (xq0, xq1), (yq0, yq1), (zq0, zq1))
        out_v = (ov0, ov1)
        sem_g = (sg0, sg1)
        sem_o = (so0, so1)
        tables = (xd_hbm, yd_hbm, zd_hbm, xq_hbm, yq_hbm, zq_hbm)

        def gather_args(s):
            for t, tab in enumerate(tables):
                idx = id_v[s] if t < 3 else iq_v[s]
                yield tab.at[idx], planes[t][s], sem_g[s]

        def fetch(g, s):
            base = wid * per_worker + g * BLOCK
            pltpu.sync_copy(nn_hbm.at[pl.ds(2 * base, 2 * BLOCK)], nn_v[s])

            def deint(j, _):
                two_rows = (j * LANES) * 2 + lane * _splat_i(2)
                iq = plsc.load_gather(nn_v[s], [two_rows])
                idd = plsc.load_gather(nn_v[s], [two_rows + _splat_i(1)])
                sl = pl.ds(j * LANES, LANES)
                iq_v[s][sl] = iq
                id_v[s][sl] = idd
                return 0

            lax.fori_loop(0, BLOCK // LANES, deint, 0)
            for src, dst, sem in gather_args(s):
                pltpu.async_copy(src, dst, sem)

        def drain_gathers(s):
            for src, dst, sem in gather_args(s):
                pltpu.make_async_copy(src, dst, sem).wait()

        def out_slice(g):
            base = wid * per_worker + g * BLOCK
            return out_hbm.at[pl.ds(base, BLOCK), :]

        UNROLL = 2

        def compute(g, s):
            xdv, ydv, zdv = planes[0][s], planes[1][s], planes[2][s]
            xqv, yqv, zqv = planes[3][s], planes[4][s], planes[5][s]
            ov = out_v[s]

            def group(jj):
                sl = pl.ds(jj * LANES, LANES)
                dx = xdv[sl] - xqv[sl]
                dy = ydv[sl] - yqv[sl]
                dz = zdv[sl] - zqv[sl]
                s2 = dx * dx + dy * dy + dz * dz
                rinv = _rsqrt_newton(s2)
                coeffs = _sh_coeffs(dx * rinv, dy * rinv, dz * rinv)
                rows = jnp.full((LANES,), jj * LANES, jnp.int32) + lane
                for c in range(16):
                    plsc.store_scatter(ov, [rows, _splat_i(c)], coeffs[c])

            def vec_body(j, _):
                for u in range(UNROLL):
                    group(j * UNROLL + u)
                return 0

            main_groups = (BLOCK // LANES) // UNROLL
            lax.fori_loop(0, main_groups, vec_body, 0)
            for jj in range(main_groups * UNROLL, BLOCK // LANES):
                group(jj)

        def block_step(g, s):
            fetch(g + 1, 1 - s)
            drain_gathers(s)

            @pl.when(g >= 2)
            def _():
                pltpu.make_async_copy(out_v[s], out_slice(g - 2), sem_o[s]).wait()

            compute(g, s)
            pltpu.async_copy(out_v[s], out_slice(g), sem_o[s])

        fetch(0, 0)

        def pair_body(i, _):
            block_step(2 * i, 0)
            block_step(2 * i + 1, 1)
            return 0

        lax.fori_loop(0, (nblocks - 1) // 2, pair_body, 0)
        g_last = nblocks - 1
        drain_gathers(0)
        pltpu.make_async_copy(out_v[0], out_slice(g_last - 2), sem_o[0]).wait()
        compute(g_last, 0)
        pltpu.async_copy(out_v[0], out_slice(g_last), sem_o[0])
        pltpu.make_async_copy(out_v[1], out_slice(g_last - 1), sem_o[1]).wait()
        pltpu.make_async_copy(out_v[0], out_slice(g_last), sem_o[0]).wait()

    return sc_kernel(xd, yd, zd, xq, yq, zq, nn_flat)


# speedup vs baseline: 2.6348x; 2.6348x over previous
import functools
import math

import jax
import jax.numpy as jnp
from jax import lax
from jax.experimental import pallas as pl
from jax.experimental.pallas import tpu as pltpu
from jax.experimental.pallas import tpu_sc as plsc

NUM_CORES = 2
NUM_SUBCORES = 16
NUM_WORKERS = NUM_CORES * NUM_SUBCORES
LANES = 16
BLOCK = 2000

_C0 = math.sqrt(1.0 / (4.0 * math.pi))
_C1 = math.sqrt(3.0 / (4.0 * math.pi))
_C2M2 = math.sqrt(15.0 / (4.0 * math.pi))
_C20 = 0.25 * math.sqrt(5.0 / math.pi)
_C22 = 0.25 * math.sqrt(15.0 / math.pi)
_C3M3 = math.sqrt(35.0 / (32.0 * math.pi))
_C3M2 = 0.5 * math.sqrt(105.0 / math.pi)
_C3M1 = math.sqrt(21.0 / (32.0 * math.pi))
_C30 = 0.25 * math.sqrt(7.0 / math.pi)
_C32 = 0.25 * math.sqrt(105.0 / math.pi)


def _splat_f(v):
    return jnp.full((LANES,), v, jnp.float32)


def _splat_i(v):
    return jnp.full((LANES,), v, jnp.int32)


def _rsqrt_newton(s2):
    i = lax.bitcast_convert_type(s2, jnp.int32)
    seed = _splat_i(0x5F3759DF) - lax.shift_right_arithmetic(i, _splat_i(1))
    y = lax.bitcast_convert_type(seed, jnp.float32)
    half = _splat_f(0.5) * s2
    three_half = _splat_f(1.5)
    for _ in range(2):
        y = y * (three_half - half * y * y)
    return y


def _sh_coeffs(X, Y, Z):
    X2 = X * X
    Y2 = Y * Y
    Z2 = Z * Z
    XY = X * Y
    one = _splat_f(1.0)
    return [
        _splat_f(_C0),
        _splat_f(-_C1) * Y,
        _splat_f(_C1) * Z,
        _splat_f(-_C1) * X,
        _splat_f(_C2M2) * XY,
        _splat_f(-_C2M2) * (Y * Z),
        _splat_f(_C20) * (_splat_f(3.0) * Z2 - one),
        _splat_f(-_C2M2) * (X * Z),
        _splat_f(_C22) * (X2 - Y2),
        _splat_f(-_C3M3) * Y * (_splat_f(3.0) * X2 - Y2),
        _splat_f(_C3M2) * XY * Z,
        _splat_f(-_C3M1) * Y * (_splat_f(5.0) * Z2 - one),
        _splat_f(_C30) * Z * (_splat_f(5.0) * Z2 - _splat_f(3.0)),
        _splat_f(-_C3M1) * X * (_splat_f(5.0) * Z2 - one),
        _splat_f(_C32) * Z * (X2 - Y2),
        _splat_f(-_C3M3) * X * (X2 - _splat_f(3.0) * Y2),
    ]


def _column(arr, c):
    onehot = jnp.zeros((arr.shape[1],), arr.dtype).at[c].set(1)
    return jnp.sum(arr * onehot[None, :], axis=1)


def kernel(xyz_data, xyz_query, nn_idx):
    num_edges = nn_idx.shape[0]
    per_worker = num_edges // NUM_WORKERS
    assert per_worker * NUM_WORKERS == num_edges
    assert per_worker % BLOCK == 0
    nblocks = per_worker // BLOCK
    assert nblocks >= 3 and nblocks % 2 == 1

    xd, yd, zd = (_column(xyz_data, c) for c in range(3))
    xq, yq, zq = (_column(xyz_query, c) for c in range(3))
    idx_q = _column(nn_idx, 0)
    idx_d = _column(nn_idx, 1)

    mesh = plsc.VectorSubcoreMesh(core_axis_name="c", subcore_axis_name="s")

    scratch = (
        [pltpu.VMEM((BLOCK,), jnp.int32)] * 4
        + [pltpu.VMEM((BLOCK,), jnp.float32)] * 12
        + [pltpu.VMEM((BLOCK, 16), jnp.float32)] * 2
        + [pltpu.SemaphoreType.DMA] * 4
    )

    @functools.partial(
        pl.kernel,
        out_type=jax.ShapeDtypeStruct((num_edges, 16), jnp.float32),
        mesh=mesh,
        scratch_types=scratch,
        compiler_params=pltpu.CompilerParams(
            needs_layout_passes=False, use_tc_tiling_on_sc=False
        ),
    )
    def sc_kernel(
        xd_hbm, yd_hbm, zd_hbm, xq_hbm, yq_hbm, zq_hbm, iq_hbm, id_hbm, out_hbm,
        iq0, iq1, id0, id1,
        xd0, xd1, yd0, yd1, zd0, zd1, xq0, xq1, yq0, yq1, zq0, zq1,
        ov0, ov1,
        sg0, sg1, so0, so1,
    ):
        wid = lax.axis_index("s") * NUM_CORES + lax.axis_index("c")
        lane = lax.iota(jnp.int32, 16)
        iq_v = (iq0, iq1)
        id_v = (id0, id1)
        planes = ((xd0, xd1), (yd0, yd1), (zd0, zd1),
                  (xq0, xq1), (yq0, yq1), (zq0, zq1))
        out_v = (ov0, ov1)
        sem_g = (sg0, sg1)
        sem_o = (so0, so1)
        tables = (xd_hbm, yd_hbm, zd_hbm, xq_hbm, yq_hbm, zq_hbm)

        def gather_args(s):
            for t, tab in enumerate(tables):
                idx = id_v[s] if t < 3 else iq_v[s]
                yield tab.at[idx], planes[t][s], sem_g[s]

        def fetch(g, s):
            base = wid * per_worker + g * BLOCK
            pltpu.sync_copy(iq_hbm.at[pl.ds(base, BLOCK)], iq_v[s])
            pltpu.sync_copy(id_hbm.at[pl.ds(base, BLOCK)], id_v[s])
            for src, dst, sem in gather_args(s):
                pltpu.async_copy(src, dst, sem)

        def drain_gathers(s):
            for src, dst, sem in gather_args(s):
                pltpu.make_async_copy(src, dst, sem).wait()

        def out_slice(g):
            base = wid * per_worker + g * BLOCK
            return out_hbm.at[pl.ds(base, BLOCK), :]

        UNROLL = 2

        def compute(g, s):
            xdv, ydv, zdv = planes[0][s], planes[1][s], planes[2][s]
            xqv, yqv, zqv = planes[3][s], planes[4][s], planes[5][s]
            ov = out_v[s]

            def group(jj):
                sl = pl.ds(jj * LANES, LANES)
                dx = xdv[sl] - xqv[sl]
                dy = ydv[sl] - yqv[sl]
                dz = zdv[sl] - zqv[sl]
                s2 = dx * dx + dy * dy + dz * dz
                rinv = _rsqrt_newton(s2)
                coeffs = _sh_coeffs(dx * rinv, dy * rinv, dz * rinv)
                rows = jnp.full((LANES,), jj * LANES, jnp.int32) + lane
                for c in range(16):
                    plsc.store_scatter(ov, [rows, _splat_i(c)], coeffs[c])

            def vec_body(j, _):
                for u in range(UNROLL):
                    group(j * UNROLL + u)
                return 0

            main_groups = (BLOCK // LANES) // UNROLL
            lax.fori_loop(0, main_groups, vec_body, 0)
            for jj in range(main_groups * UNROLL, BLOCK // LANES):
                group(jj)

        def block_step(g, s):
            fetch(g + 1, 1 - s)
            drain_gathers(s)

            @pl.when(g >= 2)
            def _():
                pltpu.make_async_copy(out_v[s], out_slice(g - 2), sem_o[s]).wait()

            compute(g, s)
            pltpu.async_copy(out_v[s], out_slice(g), sem_o[s])

        fetch(0, 0)

        def pair_body(i, _):
            block_step(2 * i, 0)
            block_step(2 * i + 1, 1)
            return 0

        lax.fori_loop(0, (nblocks - 1) // 2, pair_body, 0)
        g_last = nblocks - 1
        drain_gathers(0)
        pltpu.make_async_copy(out_v[0], out_slice(g_last - 2), sem_o[0]).wait()
        compute(g_last, 0)
        pltpu.async_copy(out_v[0], out_slice(g_last), sem_o[0])
        pltpu.make_async_copy(out_v[1], out_slice(g_last - 1), sem_o[1]).wait()
        pltpu.make_async_copy(out_v[0], out_slice(g_last), sem_o[0]).wait()

    return sc_kernel(xd, yd, zd, xq, yq, zq, idx_q, idx_d)
